# baseline (speedup 1.0000x reference)
import functools

import jax
import jax.numpy as jnp
from jax import lax
from jax.experimental import pallas as pl
from jax.experimental.pallas import tpu as pltpu
from jax.experimental.pallas import tpu_sc as plsc

NW = 32
CH = 128
L = 16


def _mf_body(n_ch, b_per_w, K, u_hbm, i_hbm, w_hbm, h_hbm, out_hbm,
             idx_u, idx_v, ru, rv, ov, sem):
    wid = lax.axis_index("s") * 2 + lax.axis_index("c")
    base = wid * b_per_w

    pltpu.sync_copy(u_hbm.at[wid], idx_u)
    pltpu.sync_copy(i_hbm.at[wid], idx_v)

    copies = []
    for j in range(n_ch):
        copies.append(
            pltpu.async_copy(w_hbm.at[idx_u.at[j]], ru.at[pl.ds(j * CH, CH)], sem))
        copies.append(
            pltpu.async_copy(h_hbm.at[idx_v.at[j]], rv.at[pl.ds(j * CH, CH)], sem))
    for c in copies:
        c.wait()

    lane = lax.iota(jnp.int32, L)
    masks = [lane == r for r in range(L)]

    def blk(b, _):
        acc = jnp.zeros((L,), jnp.float32)
        for r in range(L):
            p = ru[b * L + r, :] * rv[b * L + r, :]
            s = jnp.sum(p)
            acc = jnp.where(masks[r], jnp.full((L,), s), acc)
        ov[pl.ds(b * L, L)] = 1.0 / (1.0 + jnp.exp(-acc))
        return 0

    lax.fori_loop(0, b_per_w // L, blk, 0)

    pltpu.sync_copy(ov, out_hbm.at[pl.ds(base, b_per_w)])


def kernel(x, W, H):
    B = x.shape[0]
    K = W.shape[1]
    b_per_w = B // NW
    n_ch = b_per_w // CH

    user = x[:, 0].astype(jnp.int32).reshape(NW, n_ch, CH)
    item = x[:, 1].astype(jnp.int32).reshape(NW, n_ch, CH)

    mesh = plsc.VectorSubcoreMesh(core_axis_name="c", subcore_axis_name="s")
    body = functools.partial(_mf_body, n_ch, b_per_w, K)
    fn = pl.kernel(
        body,
        out_type=jax.ShapeDtypeStruct((B,), jnp.float32),
        mesh=mesh,
        compiler_params=pltpu.CompilerParams(
            needs_layout_passes=False, use_tc_tiling_on_sc=False),
        scratch_types=[
            pltpu.VMEM((n_ch, CH), jnp.int32),
            pltpu.VMEM((n_ch, CH), jnp.int32),
            pltpu.VMEM((b_per_w, K), jnp.float32),
            pltpu.VMEM((b_per_w, K), jnp.float32),
            pltpu.VMEM((b_per_w,), jnp.float32),
            pltpu.SemaphoreType.DMA,
        ],
    )
    return fn(user, item, W, H)


# speedup vs baseline: 1.8299x; 1.8299x over previous
import functools

import jax
import jax.numpy as jnp
from jax import lax
from jax.experimental import pallas as pl
from jax.experimental.pallas import tpu as pltpu
from jax.experimental.pallas import tpu_sc as plsc

L = 16
CH = 128
KH = 8
TCHUNK = 7 * CH
NCHUNK = 112
VPAD = 100352


def _stage_table(t_hbm, t_tail, tw, bufs, tbuf, kbase, sid, V, ssem, lsem):
    Vt = V % CH
    full = (V - Vt) - 111 * TCHUNK

    def spread(buf, col, width):
        return [pltpu.async_copy(
            buf.at[k].at[pl.ds(0, width)],
            tw.at[pl.ds(k * VPAD + col, width)], lsem) for k in range(KH)]

    cols = [jnp.minimum(sid * 7 + c, 110) * TCHUNK for c in range(7)]
    fetch = [None] * 7
    loc = {0: [], 1: []}
    fetch[0] = pltpu.async_copy(
        t_hbm.at[pl.ds(kbase, KH), pl.ds(cols[0], TCHUNK)], bufs[0], ssem)
    for c in range(7):
        b = c & 1
        if c + 1 < 7:
            nb = (c + 1) & 1
            for x in loc[nb]:
                x.wait()
            loc[nb] = []
            fetch[c + 1] = pltpu.async_copy(
                t_hbm.at[pl.ds(kbase, KH), pl.ds(cols[c + 1], TCHUNK)],
                bufs[nb], ssem)
        fetch[c].wait()
        loc[b] = spread(bufs[b], cols[c], TCHUNK)
    for b in (0, 1):
        for x in loc[b]:
            x.wait()

    @pl.when(sid == 15)
    def _():
        pltpu.async_copy(
            t_hbm.at[pl.ds(kbase, KH), pl.ds(111 * TCHUNK, full)],
            bufs[0].at[:, pl.ds(0, full)], ssem).wait()
        pltpu.async_copy(t_tail.at[pl.ds(kbase, KH), :], tbuf, ssem).wait()
        tail_copies = []
        for k in range(KH):
            tail_copies.append(pltpu.async_copy(
                bufs[0].at[k].at[pl.ds(0, full)],
                tw.at[pl.ds(k * VPAD + 111 * TCHUNK, full)], lsem))
            tail_copies.append(pltpu.async_copy(
                tbuf.at[k],
                tw.at[pl.ds(k * VPAD + (V - Vt), Vt)], lsem))
        for c in tail_copies:
            c.wait()


def _partial_body(V, b_per_sw, u_hbm, i_hbm, wt_hbm, ht_hbm, wtl_hbm,
                  htl_hbm, p0_hbm, p1_hbm, idx_u, idx_v, tw, th,
                  b0, b1, tbuf, gu, gv, acc, ssem, lsem, gsem):
    cid = lax.axis_index("c")
    sid = lax.axis_index("s")
    base = sid * b_per_sw
    kbase = cid * KH
    n_blk = b_per_sw // L
    bufs = [b0, b1]

    _stage_table(wt_hbm, wtl_hbm, tw, bufs, tbuf, kbase, sid, V, ssem, lsem)
    _stage_table(ht_hbm, htl_hbm, th, bufs, tbuf, kbase, sid, V, ssem, lsem)
    plsc.subcore_barrier()

    pltpu.sync_copy(u_hbm.at[pl.ds(base, b_per_sw)], idx_u)
    pltpu.sync_copy(i_hbm.at[pl.ds(base, b_per_sw)], idx_v)

    def zero(b, _):
        acc[pl.ds(b * L, L)] = jnp.zeros((L,), jnp.float32)
        return 0

    lax.fori_loop(0, n_blk, zero, 0)

    def kstep(k, _):
        tw_k = tw.at[pl.ds(k * VPAD, VPAD)]
        th_k = th.at[pl.ds(k * VPAD, VPAD)]
        gcopies = []
        for j in range(b_per_sw // CH):
            sl = pl.ds(j * CH, CH)
            gcopies.append(pltpu.async_copy(
                tw_k.at[idx_u.at[sl]], gu.at[sl], gsem))
            gcopies.append(pltpu.async_copy(
                th_k.at[idx_v.at[sl]], gv.at[sl], gsem))
        for c in gcopies:
            c.wait()

        def mac(b, _):
            sl = pl.ds(b * L, L)
            acc[sl] = acc[sl] + gu[sl] * gv[sl]
            return 0

        lax.fori_loop(0, n_blk, mac, 0)
        return 0

    lax.fori_loop(0, KH, kstep, 0)

    @pl.when(cid == 0)
    def _():
        pltpu.sync_copy(acc, p0_hbm.at[pl.ds(base, b_per_sw)])

    @pl.when(cid == 1)
    def _():
        pltpu.sync_copy(acc, p1_hbm.at[pl.ds(base, b_per_sw)])


def _combine_body(b_per_w, p0_hbm, p1_hbm, out_hbm, v0, v1, sem):
    wid = lax.axis_index("s") * 2 + lax.axis_index("c")
    base = wid * b_per_w
    c0 = pltpu.async_copy(p0_hbm.at[pl.ds(base, b_per_w)], v0, sem)
    c1 = pltpu.async_copy(p1_hbm.at[pl.ds(base, b_per_w)], v1, sem)
    c0.wait()
    c1.wait()

    def sig(b, _):
        sl = pl.ds(b * L, L)
        z = v0[sl] + v1[sl]
        v0[sl] = 1.0 / (1.0 + jnp.exp(-z))
        return 0

    lax.fori_loop(0, b_per_w // L, sig, 0)
    pltpu.sync_copy(v0, out_hbm.at[pl.ds(base, b_per_w)])


def kernel(x, W, H):
    B = x.shape[0]
    V = W.shape[0]
    Vt = V % CH
    b_per_sw = B // 16
    b_per_w = B // 32

    user = x[:, 0].astype(jnp.int32)
    item = x[:, 1].astype(jnp.int32)
    wt = W.T
    ht = H.T
    wtail = wt[:, V - Vt:]
    htail = ht[:, V - Vt:]

    mesh = plsc.VectorSubcoreMesh(core_axis_name="c", subcore_axis_name="s")
    p_fn = pl.kernel(
        functools.partial(_partial_body, V, b_per_sw),
        out_type=(jax.ShapeDtypeStruct((B,), jnp.float32),
                  jax.ShapeDtypeStruct((B,), jnp.float32)),
        mesh=mesh,
        compiler_params=pltpu.CompilerParams(needs_layout_passes=False),
        scratch_types=[
            pltpu.VMEM((b_per_sw,), jnp.int32),
            pltpu.VMEM((b_per_sw,), jnp.int32),
            pltpu.VMEM_SHARED((KH * VPAD,), jnp.float32),
            pltpu.VMEM_SHARED((KH * VPAD,), jnp.float32),
        ] + [pltpu.VMEM((KH, TCHUNK), jnp.float32)] * 2
        + [pltpu.VMEM((KH, Vt), jnp.float32)]
        + [
            pltpu.VMEM((b_per_sw,), jnp.float32),
            pltpu.VMEM((b_per_sw,), jnp.float32),
            pltpu.VMEM((b_per_sw,), jnp.float32),
            pltpu.SemaphoreType.DMA,
            pltpu.SemaphoreType.DMA,
            pltpu.SemaphoreType.DMA,
        ],
    )
    p0, p1 = p_fn(user, item, wt, ht, wtail, htail)

    c_fn = pl.kernel(
        functools.partial(_combine_body, b_per_w),
        out_type=jax.ShapeDtypeStruct((B,), jnp.float32),
        mesh=mesh,
        compiler_params=pltpu.CompilerParams(needs_layout_passes=False),
        scratch_types=[
            pltpu.VMEM((b_per_w,), jnp.float32),
            pltpu.VMEM((b_per_w,), jnp.float32),
            pltpu.SemaphoreType.DMA,
        ],
    )
    return c_fn(p0, p1)


# speedup vs baseline: 1.9760x; 1.0798x over previous
import functools

import jax
import jax.numpy as jnp
from jax import lax
from jax.experimental import pallas as pl
from jax.experimental.pallas import tpu as pltpu
from jax.experimental.pallas import tpu_sc as plsc

L = 16
CH = 128
KH = 8
TCHUNK = 7 * CH
NCHUNK = 112
VPAD = 100352


SCH = 10 * CH
NFULL = 78


def _stage_tables(wt_hbm, ht_hbm, wtl, htl, tw, th, bufs, tbuf, kbase, sid,
                  V, ssem, lsem):
    Vt = V % CH
    tail0 = NFULL * SCH
    full = (V - Vt) - tail0

    def spread(buf, img, col, width):
        return [pltpu.async_copy(
            buf.at[k].at[pl.ds(0, width)],
            img.at[pl.ds(k * VPAD + col, width)], lsem) for k in range(KH)]

    jobs = []
    for c in range(5):
        col = jnp.minimum(sid * 5 + c, NFULL - 1) * SCH
        jobs.append((wt_hbm, tw, col))
        jobs.append((ht_hbm, th, col))

    def fire(j, buf):
        src, _, col = jobs[j]
        return pltpu.async_copy(
            src.at[pl.ds(kbase, KH), pl.ds(col, SCH)], buf, ssem)

    fetch = [fire(0, bufs[0]), fire(1, bufs[1])]
    last = {0: [], 1: []}
    for j in range(len(jobs)):
        b = j & 1
        fetch[j].wait()
        _, img, col = jobs[j]
        cur = spread(bufs[b], img, col, SCH)
        if j + 2 < len(jobs):
            for x in cur:
                x.wait()
            fetch.append(fire(j + 2, bufs[b]))
        else:
            last[b] = cur
    for b in (0, 1):
        for x in last[b]:
            x.wait()

    @pl.when(sid == 15)
    def _():
        for t_hbm, t_tail, img, buf in (
                (wt_hbm, wtl, tw, bufs[0]), (ht_hbm, htl, th, bufs[1])):
            pltpu.async_copy(
                t_hbm.at[pl.ds(kbase, KH), pl.ds(tail0, full)],
                buf.at[:, pl.ds(0, full)], ssem).wait()
            pltpu.async_copy(t_tail.at[pl.ds(kbase, KH), :], tbuf, ssem).wait()
            tail_copies = []
            for k in range(KH):
                tail_copies.append(pltpu.async_copy(
                    buf.at[k].at[pl.ds(0, full)],
                    img.at[pl.ds(k * VPAD + tail0, full)], lsem))
                tail_copies.append(pltpu.async_copy(
                    tbuf.at[k],
                    img.at[pl.ds(k * VPAD + (V - Vt), Vt)], lsem))
            for c in tail_copies:
                c.wait()


def _partial_body(V, b_per_sw, u_hbm, i_hbm, wt_hbm, ht_hbm, wtl_hbm,
                  htl_hbm, p0_hbm, p1_hbm, idx_u, idx_v, tw, th,
                  b0, b1, tbuf, gu, gv, acc, ssem, lsem, gsem):
    cid = lax.axis_index("c")
    sid = lax.axis_index("s")
    base = sid * b_per_sw
    kbase = cid * KH
    n_blk = b_per_sw // L
    bufs = [b0, b1]

    iu = pltpu.async_copy(u_hbm.at[pl.ds(base, b_per_sw)], idx_u, gsem)
    iv = pltpu.async_copy(i_hbm.at[pl.ds(base, b_per_sw)], idx_v, gsem)

    def zero(b, _):
        acc[pl.ds(b * L, L)] = jnp.zeros((L,), jnp.float32)
        return 0

    lax.fori_loop(0, n_blk, zero, 0)

    _stage_tables(wt_hbm, ht_hbm, wtl_hbm, htl_hbm, tw, th, bufs, tbuf,
                  kbase, sid, V, ssem, lsem)
    iu.wait()
    iv.wait()
    plsc.subcore_barrier()

    def kstep(k, _):
        tw_k = tw.at[pl.ds(k * VPAD, VPAD)]
        th_k = th.at[pl.ds(k * VPAD, VPAD)]
        gcopies = []
        for j in range(b_per_sw // CH):
            sl = pl.ds(j * CH, CH)
            gcopies.append(pltpu.async_copy(
                tw_k.at[idx_u.at[sl]], gu.at[sl], gsem))
            gcopies.append(pltpu.async_copy(
                th_k.at[idx_v.at[sl]], gv.at[sl], gsem))
        for c in gcopies:
            c.wait()

        def mac(b, _):
            sl = pl.ds(b * L, L)
            acc[sl] = acc[sl] + gu[sl] * gv[sl]
            return 0

        lax.fori_loop(0, n_blk, mac, 0)
        return 0

    lax.fori_loop(0, KH, kstep, 0)

    @pl.when(cid == 0)
    def _():
        pltpu.sync_copy(acc, p0_hbm.at[pl.ds(base, b_per_sw)])

    @pl.when(cid == 1)
    def _():
        pltpu.sync_copy(acc, p1_hbm.at[pl.ds(base, b_per_sw)])


def _combine_body(b_per_w, p0_hbm, p1_hbm, out_hbm, v0, v1, sem):
    wid = lax.axis_index("s") * 2 + lax.axis_index("c")
    base = wid * b_per_w
    c0 = pltpu.async_copy(p0_hbm.at[pl.ds(base, b_per_w)], v0, sem)
    c1 = pltpu.async_copy(p1_hbm.at[pl.ds(base, b_per_w)], v1, sem)
    c0.wait()
    c1.wait()

    def sig(b, _):
        sl = pl.ds(b * L, L)
        z = v0[sl] + v1[sl]
        v0[sl] = 1.0 / (1.0 + jnp.exp(-z))
        return 0

    lax.fori_loop(0, b_per_w // L, sig, 0)
    pltpu.sync_copy(v0, out_hbm.at[pl.ds(base, b_per_w)])


def kernel(x, W, H):
    B = x.shape[0]
    V = W.shape[0]
    Vt = V % CH
    b_per_sw = B // 16
    b_per_w = B // 32

    user = x[:, 0].astype(jnp.int32)
    item = x[:, 1].astype(jnp.int32)
    wt = W.T
    ht = H.T
    wtail = wt[:, V - Vt:]
    htail = ht[:, V - Vt:]

    mesh = plsc.VectorSubcoreMesh(core_axis_name="c", subcore_axis_name="s")
    p_fn = pl.kernel(
        functools.partial(_partial_body, V, b_per_sw),
        out_type=(jax.ShapeDtypeStruct((B,), jnp.float32),
                  jax.ShapeDtypeStruct((B,), jnp.float32)),
        mesh=mesh,
        compiler_params=pltpu.CompilerParams(needs_layout_passes=False),
        scratch_types=[
            pltpu.VMEM((b_per_sw,), jnp.int32),
            pltpu.VMEM((b_per_sw,), jnp.int32),
            pltpu.VMEM_SHARED((KH * VPAD,), jnp.float32),
            pltpu.VMEM_SHARED((KH * VPAD,), jnp.float32),
        ] + [pltpu.VMEM((KH, SCH), jnp.float32)] * 2
        + [pltpu.VMEM((KH, Vt), jnp.float32)]
        + [
            pltpu.VMEM((b_per_sw,), jnp.float32),
            pltpu.VMEM((b_per_sw,), jnp.float32),
            pltpu.VMEM((b_per_sw,), jnp.float32),
            pltpu.SemaphoreType.DMA,
            pltpu.SemaphoreType.DMA,
            pltpu.SemaphoreType.DMA,
        ],
    )
    p0, p1 = p_fn(user, item, wt, ht, wtail, htail)

    c_fn = pl.kernel(
        functools.partial(_combine_body, b_per_w),
        out_type=jax.ShapeDtypeStruct((B,), jnp.float32),
        mesh=mesh,
        compiler_params=pltpu.CompilerParams(needs_layout_passes=False),
        scratch_types=[
            pltpu.VMEM((b_per_w,), jnp.float32),
            pltpu.VMEM((b_per_w,), jnp.float32),
            pltpu.SemaphoreType.DMA,
        ],
    )
    return c_fn(p0, p1)


# speedup vs baseline: 2.0615x; 1.0433x over previous
import functools

import jax
import jax.numpy as jnp
from jax import lax
from jax.experimental import pallas as pl
from jax.experimental.pallas import tpu as pltpu
from jax.experimental.pallas import tpu_sc as plsc

L = 16
CH = 128
KH = 8
TCHUNK = 7 * CH
NCHUNK = 112
VPAD = 100352


SCH = 10 * CH
NFULL = 78


def _stage_tables(wt_hbm, ht_hbm, wtl, htl, tw, th, bufs, tbuf, kbase, sid,
                  V, ssem, lsem):
    Vt = V % CH
    tail0 = NFULL * SCH
    full = (V - Vt) - tail0

    def spread(buf, img, col, width):
        return [pltpu.async_copy(
            buf.at[k].at[pl.ds(0, width)],
            img.at[pl.ds(k * VPAD + col, width)], lsem) for k in range(KH)]

    jobs = []
    for c in range(5):
        col = jnp.minimum(sid * 5 + c, NFULL - 1) * SCH
        jobs.append((wt_hbm, tw, col))
        jobs.append((ht_hbm, th, col))

    def fire(j, buf):
        src, _, col = jobs[j]
        return pltpu.async_copy(
            src.at[pl.ds(kbase, KH), pl.ds(col, SCH)], buf, ssem)

    fetch = [fire(0, bufs[0]), fire(1, bufs[1])]
    last = {0: [], 1: []}
    for j in range(len(jobs)):
        b = j & 1
        fetch[j].wait()
        _, img, col = jobs[j]
        cur = spread(bufs[b], img, col, SCH)
        if j + 2 < len(jobs):
            for x in cur:
                x.wait()
            fetch.append(fire(j + 2, bufs[b]))
        else:
            last[b] = cur
    for b in (0, 1):
        for x in last[b]:
            x.wait()

    @pl.when(sid == 15)
    def _():
        for t_hbm, t_tail, img, buf in (
                (wt_hbm, wtl, tw, bufs[0]), (ht_hbm, htl, th, bufs[1])):
            pltpu.async_copy(
                t_hbm.at[pl.ds(kbase, KH), pl.ds(tail0, full)],
                buf.at[:, pl.ds(0, full)], ssem).wait()
            pltpu.async_copy(t_tail.at[pl.ds(kbase, KH), :], tbuf, ssem).wait()
            tail_copies = []
            for k in range(KH):
                tail_copies.append(pltpu.async_copy(
                    buf.at[k].at[pl.ds(0, full)],
                    img.at[pl.ds(k * VPAD + tail0, full)], lsem))
                tail_copies.append(pltpu.async_copy(
                    tbuf.at[k],
                    img.at[pl.ds(k * VPAD + (V - Vt), Vt)], lsem))
            for c in tail_copies:
                c.wait()


def _partial_body(V, b_per_sw, u_hbm, i_hbm, wt_hbm, ht_hbm, wtl_hbm,
                  htl_hbm, p0_hbm, p1_hbm, idx_u, idx_v, tw, th,
                  b0, b1, tbuf, gu, gv, gu2, gv2, acc, ssem, lsem, gsem,
                  gsem2):
    cid = lax.axis_index("c")
    sid = lax.axis_index("s")
    base = sid * b_per_sw
    kbase = cid * KH
    n_blk = b_per_sw // L
    bufs = [b0, b1]

    iu = pltpu.async_copy(u_hbm.at[pl.ds(base, b_per_sw)], idx_u, gsem)
    iv = pltpu.async_copy(i_hbm.at[pl.ds(base, b_per_sw)], idx_v, gsem)

    def zero(b, _):
        acc[pl.ds(b * L, L)] = jnp.zeros((L,), jnp.float32)
        return 0

    lax.fori_loop(0, n_blk, zero, 0)

    _stage_tables(wt_hbm, ht_hbm, wtl_hbm, htl_hbm, tw, th, bufs, tbuf,
                  kbase, sid, V, ssem, lsem)
    iu.wait()
    iv.wait()
    plsc.subcore_barrier()

    gub = [gu, gu2]
    gvb = [gv, gv2]
    gsems = [gsem, gsem2]

    def fire_k(k, s):
        tw_k = tw.at[pl.ds(k * VPAD, VPAD)]
        th_k = th.at[pl.ds(k * VPAD, VPAD)]
        gcopies = []
        for j in range(b_per_sw // CH):
            sl = pl.ds(j * CH, CH)
            gcopies.append(pltpu.async_copy(
                tw_k.at[idx_u.at[sl]], gub[s].at[sl], gsems[s]))
            gcopies.append(pltpu.async_copy(
                th_k.at[idx_v.at[sl]], gvb[s].at[sl], gsems[s]))
        return gcopies

    pend = fire_k(0, 0)
    for k in range(KH):
        s = k & 1
        nxt = fire_k(k + 1, 1 - s) if k + 1 < KH else []
        for c in pend:
            c.wait()

        def mac(b, _, s=s):
            sl = pl.ds(b * L, L)
            acc[sl] = acc[sl] + gub[s][sl] * gvb[s][sl]
            return 0

        lax.fori_loop(0, n_blk, mac, 0)
        pend = nxt

    @pl.when(cid == 0)
    def _():
        pltpu.sync_copy(acc, p0_hbm.at[pl.ds(base, b_per_sw)])

    @pl.when(cid == 1)
    def _():
        pltpu.sync_copy(acc, p1_hbm.at[pl.ds(base, b_per_sw)])


def _combine_body(b_per_w, p0_hbm, p1_hbm, out_hbm, v0, v1, sem):
    wid = lax.axis_index("s") * 2 + lax.axis_index("c")
    base = wid * b_per_w
    c0 = pltpu.async_copy(p0_hbm.at[pl.ds(base, b_per_w)], v0, sem)
    c1 = pltpu.async_copy(p1_hbm.at[pl.ds(base, b_per_w)], v1, sem)
    c0.wait()
    c1.wait()

    def sig(b, _):
        sl = pl.ds(b * L, L)
        z = v0[sl] + v1[sl]
        v0[sl] = 1.0 / (1.0 + jnp.exp(-z))
        return 0

    lax.fori_loop(0, b_per_w // L, sig, 0)
    pltpu.sync_copy(v0, out_hbm.at[pl.ds(base, b_per_w)])


def kernel(x, W, H):
    B = x.shape[0]
    V = W.shape[0]
    Vt = V % CH
    b_per_sw = B // 16
    b_per_w = B // 32

    user = x[:, 0].astype(jnp.int32)
    item = x[:, 1].astype(jnp.int32)
    wt = W.T
    ht = H.T
    wtail = wt[:, V - Vt:]
    htail = ht[:, V - Vt:]

    mesh = plsc.VectorSubcoreMesh(core_axis_name="c", subcore_axis_name="s")
    p_fn = pl.kernel(
        functools.partial(_partial_body, V, b_per_sw),
        out_type=(jax.ShapeDtypeStruct((B,), jnp.float32),
                  jax.ShapeDtypeStruct((B,), jnp.float32)),
        mesh=mesh,
        compiler_params=pltpu.CompilerParams(needs_layout_passes=False),
        scratch_types=[
            pltpu.VMEM((b_per_sw,), jnp.int32),
            pltpu.VMEM((b_per_sw,), jnp.int32),
            pltpu.VMEM_SHARED((KH * VPAD,), jnp.float32),
            pltpu.VMEM_SHARED((KH * VPAD,), jnp.float32),
        ] + [pltpu.VMEM((KH, SCH), jnp.float32)] * 2
        + [pltpu.VMEM((KH, Vt), jnp.float32)]
        + [
            pltpu.VMEM((b_per_sw,), jnp.float32),
            pltpu.VMEM((b_per_sw,), jnp.float32),
            pltpu.VMEM((b_per_sw,), jnp.float32),
            pltpu.VMEM((b_per_sw,), jnp.float32),
            pltpu.VMEM((b_per_sw,), jnp.float32),
            pltpu.SemaphoreType.DMA,
            pltpu.SemaphoreType.DMA,
            pltpu.SemaphoreType.DMA,
            pltpu.SemaphoreType.DMA,
        ],
    )
    p0, p1 = p_fn(user, item, wt, ht, wtail, htail)

    c_fn = pl.kernel(
        functools.partial(_combine_body, b_per_w),
        out_type=jax.ShapeDtypeStruct((B,), jnp.float32),
        mesh=mesh,
        compiler_params=pltpu.CompilerParams(needs_layout_passes=False),
        scratch_types=[
            pltpu.VMEM((b_per_w,), jnp.float32),
            pltpu.VMEM((b_per_w,), jnp.float32),
            pltpu.SemaphoreType.DMA,
        ],
    )
    return c_fn(p0, p1)


# speedup vs baseline: 2.0729x; 1.0055x over previous
import functools

import jax
import jax.numpy as jnp
from jax import lax
from jax.experimental import pallas as pl
from jax.experimental.pallas import tpu as pltpu
from jax.experimental.pallas import tpu_sc as plsc

L = 16
CH = 128
KH = 8
TCHUNK = 7 * CH
NCHUNK = 112
VPAD = 100352


SCH = 7 * CH
NFULL = 111


def _stage_tables(wt_hbm, ht_hbm, wtl, htl, tw, th, bufs, tbuf, kbase, sid,
                  V, ssem, lsem):
    Vt = V % CH
    tail0 = NFULL * SCH
    full = (V - Vt) - tail0

    def spread(buf, img, col, width):
        return [pltpu.async_copy(
            buf.at[k].at[pl.ds(0, width)],
            img.at[pl.ds(k * VPAD + col, width)], lsem) for k in range(KH)]

    jobs = []
    for c in range(7):
        col = jnp.minimum(sid * 7 + c, NFULL - 1) * SCH
        jobs.append((wt_hbm, tw, col))
        jobs.append((ht_hbm, th, col))
    nj = len(jobs)
    nb = len(bufs)

    def fire(j, buf):
        src, _, col = jobs[j]
        return pltpu.async_copy(
            src.at[pl.ds(kbase, KH), pl.ds(col, SCH)], buf, ssem)

    fetch = [fire(j, bufs[j]) for j in range(nb)]
    last = {b: [] for b in range(nb)}
    for j in range(nj):
        b = j % nb
        fetch[j].wait()
        _, img, col = jobs[j]
        cur = spread(bufs[b], img, col, SCH)
        if j + nb < nj:
            for x in cur:
                x.wait()
            fetch.append(fire(j + nb, bufs[b]))
        else:
            last[b] = cur
    for b in range(nb):
        for x in last[b]:
            x.wait()

    @pl.when(sid == 15)
    def _():
        for t_hbm, t_tail, img, buf in (
                (wt_hbm, wtl, tw, bufs[0]), (ht_hbm, htl, th, bufs[1])):
            pltpu.async_copy(
                t_hbm.at[pl.ds(kbase, KH), pl.ds(tail0, full)],
                buf.at[:, pl.ds(0, full)], ssem).wait()
            pltpu.async_copy(t_tail.at[pl.ds(kbase, KH), :], tbuf, ssem).wait()
            tail_copies = []
            for k in range(KH):
                tail_copies.append(pltpu.async_copy(
                    buf.at[k].at[pl.ds(0, full)],
                    img.at[pl.ds(k * VPAD + tail0, full)], lsem))
                tail_copies.append(pltpu.async_copy(
                    tbuf.at[k],
                    img.at[pl.ds(k * VPAD + (V - Vt), Vt)], lsem))
            for c in tail_copies:
                c.wait()


def _partial_body(V, b_per_sw, u_hbm, i_hbm, wt_hbm, ht_hbm, wtl_hbm,
                  htl_hbm, p0_hbm, p1_hbm, idx_u, idx_v, tw, th,
                  b0, b1, b2, tbuf, gu, gv, gu2, gv2, acc, ssem, lsem, gsem,
                  gsem2):
    cid = lax.axis_index("c")
    sid = lax.axis_index("s")
    base = sid * b_per_sw
    kbase = cid * KH
    n_blk = b_per_sw // L
    bufs = [b0, b1, b2]

    iu = pltpu.async_copy(u_hbm.at[pl.ds(base, b_per_sw)], idx_u, gsem)
    iv = pltpu.async_copy(i_hbm.at[pl.ds(base, b_per_sw)], idx_v, gsem)

    def zero(b, _):
        acc[pl.ds(b * L, L)] = jnp.zeros((L,), jnp.float32)
        return 0

    lax.fori_loop(0, n_blk, zero, 0)

    _stage_tables(wt_hbm, ht_hbm, wtl_hbm, htl_hbm, tw, th, bufs, tbuf,
                  kbase, sid, V, ssem, lsem)
    iu.wait()
    iv.wait()
    plsc.subcore_barrier()

    gub = [gu, gu2]
    gvb = [gv, gv2]
    gsems = [gsem, gsem2]

    def fire_k(k, s):
        tw_k = tw.at[pl.ds(k * VPAD, VPAD)]
        th_k = th.at[pl.ds(k * VPAD, VPAD)]
        gcopies = []
        for j in range(b_per_sw // CH):
            sl = pl.ds(j * CH, CH)
            gcopies.append(pltpu.async_copy(
                tw_k.at[idx_u.at[sl]], gub[s].at[sl], gsems[s]))
            gcopies.append(pltpu.async_copy(
                th_k.at[idx_v.at[sl]], gvb[s].at[sl], gsems[s]))
        return gcopies

    pend = fire_k(0, 0)
    for k in range(KH):
        s = k & 1
        nxt = fire_k(k + 1, 1 - s) if k + 1 < KH else []
        for c in pend:
            c.wait()

        def mac(b, _, s=s):
            sl = pl.ds(b * L, L)
            acc[sl] = acc[sl] + gub[s][sl] * gvb[s][sl]
            return 0

        lax.fori_loop(0, n_blk, mac, 0)
        pend = nxt

    @pl.when(cid == 0)
    def _():
        pltpu.sync_copy(acc, p0_hbm.at[pl.ds(base, b_per_sw)])

    @pl.when(cid == 1)
    def _():
        pltpu.sync_copy(acc, p1_hbm.at[pl.ds(base, b_per_sw)])


def _combine_body(b_per_w, p0_hbm, p1_hbm, out_hbm, v0, v1, sem):
    wid = lax.axis_index("s") * 2 + lax.axis_index("c")
    base = wid * b_per_w
    c0 = pltpu.async_copy(p0_hbm.at[pl.ds(base, b_per_w)], v0, sem)
    c1 = pltpu.async_copy(p1_hbm.at[pl.ds(base, b_per_w)], v1, sem)
    c0.wait()
    c1.wait()

    def sig(b, _):
        sl = pl.ds(b * L, L)
        z = v0[sl] + v1[sl]
        v0[sl] = 1.0 / (1.0 + jnp.exp(-z))
        return 0

    lax.fori_loop(0, b_per_w // L, sig, 0)
    pltpu.sync_copy(v0, out_hbm.at[pl.ds(base, b_per_w)])


def kernel(x, W, H):
    B = x.shape[0]
    V = W.shape[0]
    Vt = V % CH
    b_per_sw = B // 16
    b_per_w = B // 32

    user = x[:, 0].astype(jnp.int32)
    item = x[:, 1].astype(jnp.int32)
    wt = W.T
    ht = H.T
    wtail = wt[:, V - Vt:]
    htail = ht[:, V - Vt:]

    mesh = plsc.VectorSubcoreMesh(core_axis_name="c", subcore_axis_name="s")
    p_fn = pl.kernel(
        functools.partial(_partial_body, V, b_per_sw),
        out_type=(jax.ShapeDtypeStruct((B,), jnp.float32),
                  jax.ShapeDtypeStruct((B,), jnp.float32)),
        mesh=mesh,
        compiler_params=pltpu.CompilerParams(needs_layout_passes=False),
        scratch_types=[
            pltpu.VMEM((b_per_sw,), jnp.int32),
            pltpu.VMEM((b_per_sw,), jnp.int32),
            pltpu.VMEM_SHARED((KH * VPAD,), jnp.float32),
            pltpu.VMEM_SHARED((KH * VPAD,), jnp.float32),
        ] + [pltpu.VMEM((KH, SCH), jnp.float32)] * 3
        + [pltpu.VMEM((KH, Vt), jnp.float32)]
        + [
            pltpu.VMEM((b_per_sw,), jnp.float32),
            pltpu.VMEM((b_per_sw,), jnp.float32),
            pltpu.VMEM((b_per_sw,), jnp.float32),
            pltpu.VMEM((b_per_sw,), jnp.float32),
            pltpu.VMEM((b_per_sw,), jnp.float32),
            pltpu.SemaphoreType.DMA,
            pltpu.SemaphoreType.DMA,
            pltpu.SemaphoreType.DMA,
            pltpu.SemaphoreType.DMA,
        ],
    )
    p0, p1 = p_fn(user, item, wt, ht, wtail, htail)

    c_fn = pl.kernel(
        functools.partial(_combine_body, b_per_w),
        out_type=jax.ShapeDtypeStruct((B,), jnp.float32),
        mesh=mesh,
        compiler_params=pltpu.CompilerParams(needs_layout_passes=False),
        scratch_types=[
            pltpu.VMEM((b_per_w,), jnp.float32),
            pltpu.VMEM((b_per_w,), jnp.float32),
            pltpu.SemaphoreType.DMA,
        ],
    )
    return c_fn(p0, p1)


# speedup vs baseline: 2.0763x; 1.0016x over previous
import functools

import jax
import jax.numpy as jnp
from jax import lax
from jax.experimental import pallas as pl
from jax.experimental.pallas import tpu as pltpu
from jax.experimental.pallas import tpu_sc as plsc

L = 16
CH = 128
KH = 8
TCHUNK = 7 * CH
NCHUNK = 112
VPAD = 100352


SCH = 7 * CH
NFULL = 111


def _stage_tables(wt_hbm, ht_hbm, wtl, htl, tw, th, bufs, tbuf, kbase, sid,
                  V, ssem, lsem):
    Vt = V % CH
    tail0 = NFULL * SCH
    full = (V - Vt) - tail0

    def spread(buf, img, col, width):
        return [pltpu.async_copy(
            buf.at[k].at[pl.ds(0, width)],
            img.at[pl.ds(k * VPAD + col, width)], lsem) for k in range(KH)]

    jobs = []
    for c in range(7):
        col = jnp.minimum(sid * 7 + c, NFULL - 1) * SCH
        jobs.append((wt_hbm, tw, col))
        jobs.append((ht_hbm, th, col))
    nj = len(jobs)
    nb = len(bufs)

    def fire(j, buf):
        src, _, col = jobs[j]
        return pltpu.async_copy(
            src.at[pl.ds(kbase, KH), pl.ds(col, SCH)], buf, ssem)

    fetch = [fire(j, bufs[j]) for j in range(nb)]
    last = {b: [] for b in range(nb)}
    for j in range(nj):
        b = j % nb
        fetch[j].wait()
        _, img, col = jobs[j]
        cur = spread(bufs[b], img, col, SCH)
        if j + nb < nj:
            for x in cur:
                x.wait()
            fetch.append(fire(j + nb, bufs[b]))
        else:
            last[b] = cur
    for b in range(nb):
        for x in last[b]:
            x.wait()

    @pl.when(sid == 15)
    def _():
        for t_hbm, t_tail, img, buf in (
                (wt_hbm, wtl, tw, bufs[0]), (ht_hbm, htl, th, bufs[1])):
            pltpu.async_copy(
                t_hbm.at[pl.ds(kbase, KH), pl.ds(tail0, full)],
                buf.at[:, pl.ds(0, full)], ssem).wait()
            pltpu.async_copy(t_tail.at[pl.ds(kbase, KH), :], tbuf, ssem).wait()
            tail_copies = []
            for k in range(KH):
                tail_copies.append(pltpu.async_copy(
                    buf.at[k].at[pl.ds(0, full)],
                    img.at[pl.ds(k * VPAD + tail0, full)], lsem))
                tail_copies.append(pltpu.async_copy(
                    tbuf.at[k],
                    img.at[pl.ds(k * VPAD + (V - Vt), Vt)], lsem))
            for c in tail_copies:
                c.wait()


def _partial_body(V, b_per_sw, u_hbm, i_hbm, wt_hbm, ht_hbm, wtl_hbm,
                  htl_hbm, p0_hbm, p1_hbm, idx_u, idx_v, tw, th,
                  b0, b1, b2, tbuf, gu, gv, acc, ssem, lsem, gsem, gsem2):
    cid = lax.axis_index("c")
    sid = lax.axis_index("s")
    base = sid * b_per_sw
    kbase = cid * KH
    n_blk = b_per_sw // L
    bufs = [b0, b1, b2]

    iu = pltpu.async_copy(u_hbm.at[pl.ds(base, b_per_sw)], idx_u, gsem)
    iv = pltpu.async_copy(i_hbm.at[pl.ds(base, b_per_sw)], idx_v, gsem)

    def zero(b, _):
        acc[pl.ds(b * L, L)] = jnp.zeros((L,), jnp.float32)
        return 0

    lax.fori_loop(0, n_blk, zero, 0)

    _stage_tables(wt_hbm, ht_hbm, wtl_hbm, htl_hbm, tw, th, bufs, tbuf,
                  kbase, sid, V, ssem, lsem)
    iu.wait()
    iv.wait()
    plsc.subcore_barrier()

    def fire_k(k, buf_off, sem):
        tw_k = tw.at[pl.ds(k * VPAD, VPAD)]
        th_k = th.at[pl.ds(k * VPAD, VPAD)]
        gcopies = []
        for j in range(b_per_sw // CH):
            sl = pl.ds(j * CH, CH)
            dsl = pl.ds(buf_off + j * CH, CH)
            gcopies.append(pltpu.async_copy(
                tw_k.at[idx_u.at[sl]], gu.at[dsl], sem))
            gcopies.append(pltpu.async_copy(
                th_k.at[idx_v.at[sl]], gv.at[dsl], sem))
        return gcopies

    dummy = wt_hbm.at[0, pl.ds(0, b_per_sw)]

    def drain_and_mac(buf_off, sem):
        pltpu.make_async_copy(
            dummy, gu.at[pl.ds(buf_off, b_per_sw)], sem).wait()
        pltpu.make_async_copy(
            dummy, gv.at[pl.ds(buf_off, b_per_sw)], sem).wait()

        def mac(b, _):
            sl = pl.ds(b * L, L)
            dsl = pl.ds(buf_off + b * L, L)
            acc[sl] = acc[sl] + gu[dsl] * gv[dsl]
            return 0

        lax.fori_loop(0, n_blk, mac, 0)

    fire_k(0, 0, gsem)

    def kpair(t, _):
        k = 2 * t
        fire_k(k + 1, b_per_sw, gsem2)
        drain_and_mac(0, gsem)

        @pl.when(k + 2 < KH)
        def _():
            fire_k(k + 2, 0, gsem)

        drain_and_mac(b_per_sw, gsem2)
        return 0

    lax.fori_loop(0, KH // 2, kpair, 0)

    @pl.when(cid == 0)
    def _():
        pltpu.sync_copy(acc, p0_hbm.at[pl.ds(base, b_per_sw)])

    @pl.when(cid == 1)
    def _():
        pltpu.sync_copy(acc, p1_hbm.at[pl.ds(base, b_per_sw)])


def _combine_body(b_per_w, p0_hbm, p1_hbm, out_hbm, v0, v1, sem):
    wid = lax.axis_index("s") * 2 + lax.axis_index("c")
    base = wid * b_per_w
    c0 = pltpu.async_copy(p0_hbm.at[pl.ds(base, b_per_w)], v0, sem)
    c1 = pltpu.async_copy(p1_hbm.at[pl.ds(base, b_per_w)], v1, sem)
    c0.wait()
    c1.wait()

    def sig(b, _):
        sl = pl.ds(b * L, L)
        z = v0[sl] + v1[sl]
        v0[sl] = 1.0 / (1.0 + jnp.exp(-z))
        return 0

    lax.fori_loop(0, b_per_w // L, sig, 0)
    pltpu.sync_copy(v0, out_hbm.at[pl.ds(base, b_per_w)])


def kernel(x, W, H):
    B = x.shape[0]
    V = W.shape[0]
    Vt = V % CH
    b_per_sw = B // 16
    b_per_w = B // 32

    user = x[:, 0].astype(jnp.int32)
    item = x[:, 1].astype(jnp.int32)
    wt = W.T
    ht = H.T
    wtail = wt[:, V - Vt:]
    htail = ht[:, V - Vt:]

    mesh = plsc.VectorSubcoreMesh(core_axis_name="c", subcore_axis_name="s")
    p_fn = pl.kernel(
        functools.partial(_partial_body, V, b_per_sw),
        out_type=(jax.ShapeDtypeStruct((B,), jnp.float32),
                  jax.ShapeDtypeStruct((B,), jnp.float32)),
        mesh=mesh,
        compiler_params=pltpu.CompilerParams(needs_layout_passes=False),
        scratch_types=[
            pltpu.VMEM((b_per_sw,), jnp.int32),
            pltpu.VMEM((b_per_sw,), jnp.int32),
            pltpu.VMEM_SHARED((KH * VPAD,), jnp.float32),
            pltpu.VMEM_SHARED((KH * VPAD,), jnp.float32),
        ] + [pltpu.VMEM((KH, SCH), jnp.float32)] * 3
        + [pltpu.VMEM((KH, Vt), jnp.float32)]
        + [
            pltpu.VMEM((2 * b_per_sw,), jnp.float32),
            pltpu.VMEM((2 * b_per_sw,), jnp.float32),
            pltpu.VMEM((b_per_sw,), jnp.float32),
            pltpu.SemaphoreType.DMA,
            pltpu.SemaphoreType.DMA,
            pltpu.SemaphoreType.DMA,
            pltpu.SemaphoreType.DMA,
        ],
    )
    p0, p1 = p_fn(user, item, wt, ht, wtail, htail)

    c_fn = pl.kernel(
        functools.partial(_combine_body, b_per_w),
        out_type=jax.ShapeDtypeStruct((B,), jnp.float32),
        mesh=mesh,
        compiler_params=pltpu.CompilerParams(needs_layout_passes=False),
        scratch_types=[
            pltpu.VMEM((b_per_w,), jnp.float32),
            pltpu.VMEM((b_per_w,), jnp.float32),
            pltpu.SemaphoreType.DMA,
        ],
    )
    return c_fn(p0, p1)
